# baseline (speedup 1.0000x reference)
import functools
import jax
import jax.numpy as jnp
from jax import lax
from jax.experimental import pallas as pl
from jax.experimental.pallas import tpu as pltpu
from jax.experimental.pallas import tpu_sc as plsc

DIM = 128
BATCH = 16384
NUM_CORES = 2
NUM_SUBCORES = 16
NUM_WORKERS = NUM_CORES * NUM_SUBCORES
B_PER_W = BATCH // NUM_WORKERS
IDX_CHUNK = 128
N_CHUNKS = B_PER_W // IDX_CHUNK

_mesh = plsc.VectorSubcoreMesh(core_axis_name="c", subcore_axis_name="s")


@functools.partial(
    pl.kernel,
    mesh=_mesh,
    out_type=jax.ShapeDtypeStruct((BATCH, DIM), jnp.float32),
    scratch_types=[
        pltpu.VMEM((N_CHUNKS, IDX_CHUNK), jnp.int32),
        pltpu.VMEM((B_PER_W, DIM), jnp.float32),
    ]
    + [pltpu.SemaphoreType.DMA] * (2 * N_CHUNKS)
    + [pltpu.SemaphoreType.DMA],
)
def _gather_kernel(table_hbm, idx_hbm, out_hbm, idx_v, rows_v, *sems):
    isems = sems[:N_CHUNKS]
    gsems = sems[N_CHUNKS : 2 * N_CHUNKS]
    wsem = sems[2 * N_CHUNKS]
    wid = lax.axis_index("s") * NUM_CORES + lax.axis_index("c")
    base = wid * B_PER_W
    icopies = []
    for j in range(N_CHUNKS):
        icopies.append(
            pltpu.async_copy(
                idx_hbm.at[pl.ds(base + j * IDX_CHUNK, IDX_CHUNK)],
                idx_v.at[j],
                isems[j],
            )
        )
    gathers = []
    for j in range(N_CHUNKS):
        icopies[j].wait()
        gathers.append(
            pltpu.async_copy(
                table_hbm.at[idx_v.at[j]],
                rows_v.at[pl.ds(j * IDX_CHUNK, IDX_CHUNK)],
                gsems[j],
            )
        )
    writes = []
    for j in range(N_CHUNKS):
        gathers[j].wait()
        writes.append(
            pltpu.async_copy(
                rows_v.at[pl.ds(j * IDX_CHUNK, IDX_CHUNK)],
                out_hbm.at[pl.ds(base + j * IDX_CHUNK, IDX_CHUNK)],
                wsem,
            )
        )
    for w in writes:
        w.wait()


def kernel(center_words, in_emb):
    return _gather_kernel(in_emb, center_words.astype(jnp.int32))
